# baseline (speedup 1.0000x reference)
import functools

import jax
import jax.numpy as jnp
from jax.experimental import pallas as pl
from jax.experimental.pallas import tpu as pltpu


def _embed_kernel(x_ref, w_ref, pos_ref, b_ref, out_ref, mask_ref):
    x = x_ref[0].astype(jnp.bfloat16)
    w = w_ref[...].astype(jnp.bfloat16)
    acc = jnp.dot(x, w, preferred_element_type=jnp.float32)
    row = pos_ref[0] + b_ref[...][None, :]
    out_ref[0] = acc + row

    @pl.when(pl.program_id(0) == 0)
    def _():
        mask_ref[...] = jnp.ones(mask_ref.shape, dtype=jnp.bool_)


@functools.partial(jax.jit, static_argnames=("tb", "tn", "interpret"))
def _embed(seq_t, W_tok, pos3, b_tok, tb=2048, tn=2048, interpret=False):
    L, _, D = pos3.shape
    B, C = seq_t.shape[1], seq_t.shape[2]
    grid = (L, B // tb, D // tn)
    out, mask = pl.pallas_call(
        _embed_kernel,
        grid=grid,
        in_specs=[
            pl.BlockSpec((1, tb, C), lambda l, i, j: (l, i, 0)),
            pl.BlockSpec((C, tn), lambda l, i, j: (0, j)),
            pl.BlockSpec((1, 1, tn), lambda l, i, j: (l, 0, j)),
            pl.BlockSpec((tn,), lambda l, i, j: (j,)),
        ],
        out_specs=[
            pl.BlockSpec((1, tb, tn), lambda l, i, j: (l, i, j)),
            pl.BlockSpec((tb, L), lambda l, i, j: (i, 0)),
        ],
        out_shape=[
            jax.ShapeDtypeStruct((L, B, D), jnp.float32),
            jax.ShapeDtypeStruct((B, L), jnp.bool_),
        ],
        compiler_params=pltpu.CompilerParams(
            dimension_semantics=("arbitrary", "arbitrary", "arbitrary"),
        ),
        interpret=interpret,
    )(seq_t, W_tok, pos3, b_tok)
    return out, mask


def kernel(sequence, W_tok, b_tok, pos_table):
    B, L, C = sequence.shape
    D = W_tok.shape[1]
    seq_t = jnp.transpose(sequence, (1, 0, 2))
    pos3 = pos_table.reshape(L, 1, D)
    out_t, mask = _embed(seq_t, W_tok, pos3, b_tok)
    x = jnp.transpose(out_t, (1, 0, 2))
    return (x, mask)


# speedup vs baseline: 1.0643x; 1.0643x over previous
import functools

import jax
import jax.numpy as jnp
from jax.experimental import pallas as pl
from jax.experimental.pallas import tpu as pltpu


def _embed_kernel(x_ref, w_ref, pos_ref, b_ref, out_ref):
    x = x_ref[0].astype(jnp.bfloat16)
    w = w_ref[...].astype(jnp.bfloat16)
    acc = jnp.dot(x, w, preferred_element_type=jnp.float32)
    row = pos_ref[0] + b_ref[...][None, :]
    out_ref[0] = acc + row


@functools.partial(jax.jit, static_argnames=("tb", "tn", "interpret"))
def _embed(seq_t, W_tok, pos3, b_tok, tb=2048, tn=2048, interpret=False):
    L, _, D = pos3.shape
    B, C = seq_t.shape[1], seq_t.shape[2]
    grid = (L, B // tb, D // tn)
    out = pl.pallas_call(
        _embed_kernel,
        grid=grid,
        in_specs=[
            pl.BlockSpec((1, tb, C), lambda l, i, j: (l, i, 0)),
            pl.BlockSpec((C, tn), lambda l, i, j: (0, j)),
            pl.BlockSpec((1, 1, tn), lambda l, i, j: (l, 0, j)),
            pl.BlockSpec((tn,), lambda l, i, j: (j,)),
        ],
        out_specs=pl.BlockSpec((1, tb, tn), lambda l, i, j: (l, i, j)),
        out_shape=jax.ShapeDtypeStruct((L, B, D), jnp.float32),
        compiler_params=pltpu.CompilerParams(
            dimension_semantics=("arbitrary", "arbitrary", "arbitrary"),
        ),
        interpret=interpret,
    )(seq_t, W_tok, pos3, b_tok)
    return out


def kernel(sequence, W_tok, b_tok, pos_table):
    B, L, C = sequence.shape
    D = W_tok.shape[1]
    seq_t = jnp.transpose(sequence, (1, 0, 2))
    pos3 = pos_table.reshape(L, 1, D)
    out_t = _embed(seq_t, W_tok, pos3, b_tok)
    x = jnp.transpose(out_t, (1, 0, 2))
    mask = jnp.ones((B, L), dtype=bool)
    return (x, mask)


# speedup vs baseline: 1.0935x; 1.0274x over previous
import functools

import jax
import jax.numpy as jnp
from jax.experimental import pallas as pl
from jax.experimental.pallas import tpu as pltpu


def _embed_kernel(x_ref, w_ref, pos_ref, b_ref, out_ref):
    x = x_ref[0].astype(jnp.bfloat16)
    w = w_ref[...].astype(jnp.bfloat16)
    acc = jnp.dot(x, w, preferred_element_type=jnp.float32)
    row = pos_ref[pl.ds(pl.program_id(0), 1), :] + b_ref[...][None, :]
    out_ref[0] = acc + row


@functools.partial(jax.jit, static_argnames=("tb", "tn", "interpret"))
def _embed(seq_t, W_tok, pos_table, b_tok, tb=2048, tn=2048, interpret=False):
    L, D = pos_table.shape
    B, C = seq_t.shape[1], seq_t.shape[2]
    grid = (L, B // tb, D // tn)
    out = pl.pallas_call(
        _embed_kernel,
        grid=grid,
        in_specs=[
            pl.BlockSpec((1, tb, C), lambda l, i, j: (l, i, 0)),
            pl.BlockSpec((C, tn), lambda l, i, j: (0, j)),
            pl.BlockSpec((1, 1, tn), lambda l, i, j: (l, 0, j)),
            pl.BlockSpec((tn,), lambda l, i, j: (j,)),
        ],
        out_specs=pl.BlockSpec((1, tb, tn), lambda l, i, j: (l, i, j)),
        out_shape=jax.ShapeDtypeStruct((L, B, D), jnp.float32),
        compiler_params=pltpu.CompilerParams(
            dimension_semantics=("arbitrary", "arbitrary", "arbitrary"),
        ),
        interpret=interpret,
    )(seq_t, W_tok, pos_table, b_tok)
    return out


def kernel(sequence, W_tok, b_tok, pos_table):
    B, L, C = sequence.shape
    D = W_tok.shape[1]
    seq_t = jnp.transpose(sequence, (1, 0, 2))
    out_t = _embed(seq_t, W_tok, pos_table, b_tok)
    x = jnp.transpose(out_t, (1, 0, 2))
    mask = jnp.ones((B, L), dtype=bool)
    return (x, mask)
